# baseline (speedup 1.0000x reference)
import functools

import jax
import jax.numpy as jnp
from jax import lax
from jax.experimental import pallas as pl
from jax.experimental.pallas import tpu as pltpu
from jax.experimental.pallas import tpu_sc as plsc

_L = 16


def _make_seg_sum(S, A, F, H):
    NC, NS = 2, 16
    NW = NC * NS
    assert S % NW == 0
    SEG_W = S // NW
    KH = H // _L

    mesh = plsc.VectorSubcoreMesh(core_axis_name="c", subcore_axis_name="s")

    @functools.partial(
        pl.kernel,
        mesh=mesh,
        out_type=jax.ShapeDtypeStruct((S, H), jnp.float32),
        scratch_types=[
            pltpu.VMEM((SEG_W, A), jnp.int32),
            pltpu.VMEM((2, A, H // 2), jnp.int32),
            pltpu.VMEM((SEG_W, H), jnp.float32),
            pltpu.VMEM((2 * _L,), jnp.float32),
            pltpu.VMEM((2 * _L,), jnp.int32),
            pltpu.SemaphoreType.DMA,
            pltpu.SemaphoreType.DMA,
        ],
        compiler_params=pltpu.CompilerParams(needs_layout_passes=False),
    )
    def seg_sum(idx_hbm, table_hbm, out_hbm, idx_v, rows_v, out_v, w_v, idx_s,
                sem0, sem1):
        wid = lax.axis_index("s") * NC + lax.axis_index("c")
        base = wid * SEG_W
        pltpu.sync_copy(idx_hbm.at[pl.ds(base, SEG_W)], idx_v)

        def gather_desc(j, buf, sem):
            return pltpu.make_async_copy(
                table_hbm.at[idx_v.at[j]], rows_v.at[buf], sem)

        gather_desc(0, 0, sem0).start()
        gather_desc(1, 1, sem1).start()

        lane = lax.iota(jnp.int32, _L)

        himask = jnp.full((_L,), -65536, jnp.int32)

        def compute_seg(j, buf, sem):
            u = idx_v[j, pl.ds(0, _L)]
            v = idx_v[j, pl.ds(_L, _L)]
            idx_s[pl.ds(0, _L)] = u
            idx_s[pl.ds(_L, _L)] = v
            cnt_u = jnp.ones((_L,), jnp.int32)
            cnt_v = jnp.ones((_L,), jnp.int32)
            for s in range(1, _L):
                rot = (lane + s) & (_L - 1)
                cnt_u += (u == plsc.load_gather(idx_s, [rot])).astype(jnp.int32)
                cnt_v += (v == plsc.load_gather(idx_s, [rot + _L])).astype(jnp.int32)
            for s in range(_L):
                rot = (lane + s) & (_L - 1)
                cnt_u += (u == plsc.load_gather(idx_s, [rot + _L])).astype(jnp.int32)
                cnt_v += (v == plsc.load_gather(idx_s, [rot])).astype(jnp.int32)
            w_v[pl.ds(0, _L)] = 1.0 / cnt_u.astype(jnp.float32)
            w_v[pl.ds(_L, _L)] = 1.0 / cnt_v.astype(jnp.float32)
            gather_desc(j, buf, sem).wait()

            def acc_body(a, acc):
                wb = plsc.load_gather(w_v, [jnp.full((_L,), 0, jnp.int32) + a])
                out = []
                for k2 in range(KH // 2):
                    pair = rows_v[buf, a, pl.ds(k2 * _L, _L)]
                    lo = plsc.bitcast(pair << 16, jnp.float32)
                    hi = plsc.bitcast(pair & himask, jnp.float32)
                    out.append(acc[2 * k2] + wb * lo)
                    out.append(acc[2 * k2 + 1] + wb * hi)
                return tuple(out)

            acc = lax.fori_loop(
                0, A, acc_body,
                tuple(jnp.zeros((_L,), jnp.float32) for _ in range(KH)))
            for k in range(KH):
                out_v[j, pl.ds(k * _L, _L)] = acc[k]

            @pl.when(j + 2 < SEG_W)
            def _():
                gather_desc(j + 2, buf, sem).start()

        def body2(i, carry):
            compute_seg(2 * i, 0, sem0)
            compute_seg(2 * i + 1, 1, sem1)
            return carry

        lax.fori_loop(0, SEG_W // 2, body2, 0)
        pltpu.sync_copy(out_v, out_hbm.at[pl.ds(base, SEG_W)])

    return seg_sum


def _mlp_body(xw_ref, xb_ref, w1_ref, b1_ref, w2_ref, b2_ref, w3_ref, b3_ref,
              wo_ref, bo_ref, o_ref):
    dn = (((1,), (1,)), ((), ()))
    xw = jnp.clip(xw_ref[...], -1.0, 1.0)
    xb = jnp.clip(xb_ref[...], -1.0, 1.0)
    h = jnp.concatenate([xw, xb], axis=1)
    h = jnp.maximum(
        lax.dot_general(h, w1_ref[...], dn, preferred_element_type=jnp.float32)
        + b1_ref[...], 0.0)
    h = jnp.maximum(
        lax.dot_general(h, w2_ref[...], dn, preferred_element_type=jnp.float32)
        + b2_ref[...], 0.0)
    h = jnp.maximum(
        lax.dot_general(h, w3_ref[...], dn, preferred_element_type=jnp.float32)
        + b3_ref[...], 0.0)
    o_ref[...] = jnp.sum(h * wo_ref[...], axis=1, keepdims=True) + bo_ref[...]


def _mlp(acc, W1, b1, W2, b2, W3, b3, W_out, b_out):
    S, H = acc.shape
    B = S // 2
    BB = 256
    NB = B // BB
    H2, H4 = W2.shape[0], W3.shape[0]
    full = lambda shape: pl.BlockSpec(shape, lambda i: (0, 0))
    return pl.pallas_call(
        _mlp_body,
        grid=(NB,),
        in_specs=[
            pl.BlockSpec((BB, H), lambda i: (i, 0)),
            pl.BlockSpec((BB, H), lambda i: (i + NB, 0)),
            full((H, 2 * H)),
            full((1, H)),
            full((H2, H)),
            full((1, H2)),
            full((H4, H2)),
            full((1, H4)),
            full((1, H4)),
            full((1, 1)),
        ],
        out_specs=pl.BlockSpec((BB, 1), lambda i: (i, 0)),
        out_shape=jax.ShapeDtypeStruct((B, 1), jnp.float32),
    )(acc, acc, W1, b1.reshape(1, -1), W2, b2.reshape(1, -1),
      W3, b3.reshape(1, -1), W_out, b_out.reshape(1, 1))


def kernel(white_indices, black_indices, W_ft, W1, b1, W2, b2, W3, b3, W_out, b_out):
    B, A = white_indices.shape
    H, F = W_ft.shape
    idx_all = jnp.concatenate([white_indices, black_indices], axis=0)
    table = (W_ft.T.reshape(F, H // 32, 2, _L).swapaxes(2, 3)
             .reshape(F, H // 2, 2).astype(jnp.bfloat16))
    table = jax.lax.bitcast_convert_type(table, jnp.int32)
    seg_sum = _make_seg_sum(2 * B, A, F, H)
    acc = seg_sum(idx_all, table)
    out = _mlp(acc, W1, b1, W2, b2, W3, b3, W_out, b_out)
    return out[:, 0]


# speedup vs baseline: 1.5498x; 1.5498x over previous
import functools

import jax
import jax.numpy as jnp
from jax import lax
from jax.experimental import pallas as pl
from jax.experimental.pallas import tpu as pltpu
from jax.experimental.pallas import tpu_sc as plsc

_L = 16


def _make_seg_sum(S, A, F, H, G=4):
    NC, NS = 2, 16
    NW = NC * NS
    assert S % (NW * G) == 0
    SEG_W = S // NW
    GB = SEG_W // G
    GA = G * A
    KH = H // _L

    mesh = plsc.VectorSubcoreMesh(core_axis_name="c", subcore_axis_name="s")

    @functools.partial(
        pl.kernel,
        mesh=mesh,
        out_type=jax.ShapeDtypeStruct((S, H), jnp.float32),
        scratch_types=[
            pltpu.VMEM((GB, GA), jnp.int32),
            pltpu.VMEM((2, GA, H // 2), jnp.int32),
            pltpu.VMEM((SEG_W, H), jnp.float32),
            pltpu.VMEM((2 * _L * G,), jnp.float32),
            pltpu.VMEM((2 * _L * G,), jnp.int32),
            pltpu.SemaphoreType.DMA,
            pltpu.SemaphoreType.DMA,
        ],
        compiler_params=pltpu.CompilerParams(needs_layout_passes=False),
    )
    def seg_sum(idx_hbm, table_hbm, out_hbm, idx_v, rows_v, out_v, w_v, idx_s,
                sem0, sem1):
        wid = lax.axis_index("s") * NC + lax.axis_index("c")
        pltpu.sync_copy(idx_hbm.at[pl.ds(wid * GB, GB)], idx_v)

        def gather_desc(g, buf, sem):
            return pltpu.make_async_copy(
                table_hbm.at[idx_v.at[g]], rows_v.at[buf], sem)

        gather_desc(0, 0, sem0).start()
        gather_desc(1, 1, sem1).start()

        lane = lax.iota(jnp.int32, _L)
        himask = jnp.full((_L,), -65536, jnp.int32)

        def compute_group(g, buf, sem):
            for t in range(G):
                u = idx_v[g, pl.ds(t * A, _L)]
                v = idx_v[g, pl.ds(t * A + _L, _L)]
                idx_s[pl.ds(t * A, _L)] = u
                idx_s[pl.ds(t * A + _L, _L)] = v
                cnt_u = jnp.ones((_L,), jnp.int32)
                cnt_v = jnp.ones((_L,), jnp.int32)
                for s in range(1, _L):
                    rot = ((lane + s) & (_L - 1)) + t * A
                    cnt_u += (u == plsc.load_gather(idx_s, [rot])).astype(jnp.int32)
                    cnt_v += (v == plsc.load_gather(idx_s, [rot + _L])).astype(jnp.int32)
                for s in range(_L):
                    rot = ((lane + s) & (_L - 1)) + t * A
                    cnt_u += (u == plsc.load_gather(idx_s, [rot + _L])).astype(jnp.int32)
                    cnt_v += (v == plsc.load_gather(idx_s, [rot])).astype(jnp.int32)
                w_v[pl.ds(t * A, _L)] = 1.0 / cnt_u.astype(jnp.float32)
                w_v[pl.ds(t * A + _L, _L)] = 1.0 / cnt_v.astype(jnp.float32)
            gather_desc(g, buf, sem).wait()

            for t in range(G):
                def acc_body(a, acc, t=t):
                    wb = plsc.load_gather(
                        w_v, [jnp.full((_L,), t * A, jnp.int32) + a])
                    out = []
                    for k2 in range(KH // 2):
                        pair = rows_v[buf, t * A + a, pl.ds(k2 * _L, _L)]
                        lo = plsc.bitcast(pair << 16, jnp.float32)
                        hi = plsc.bitcast(pair & himask, jnp.float32)
                        out.append(acc[2 * k2] + wb * lo)
                        out.append(acc[2 * k2 + 1] + wb * hi)
                    return tuple(out)

                acc = lax.fori_loop(
                    0, A, acc_body,
                    tuple(jnp.zeros((_L,), jnp.float32) for _ in range(KH)))
                for k in range(KH):
                    out_v[g * G + t, pl.ds(k * _L, _L)] = acc[k]

            @pl.when(g + 2 < GB)
            def _():
                gather_desc(g + 2, buf, sem).start()

        def body2(i, carry):
            compute_group(2 * i, 0, sem0)
            compute_group(2 * i + 1, 1, sem1)
            return carry

        lax.fori_loop(0, GB // 2, body2, 0)
        pltpu.sync_copy(out_v, out_hbm.at[pl.ds(wid * SEG_W, SEG_W)])

    return seg_sum


def _prep_body(x_ref, o_ref):
    xt = x_ref[...].T
    half = xt.shape[1] // 2

    def bits(z):
        return pltpu.bitcast(
            z.astype(jnp.bfloat16).astype(jnp.float32), jnp.int32)

    o_ref[...] = (
        lax.shift_right_logical(bits(xt[:, :half]), 16)
        | bits(xt[:, half:]))


def _prep_table(W_ft):
    H, F = W_ft.shape
    FB = 1024
    NB = pl.cdiv(F, FB)
    return pl.pallas_call(
        _prep_body,
        grid=(NB,),
        in_specs=[pl.BlockSpec((H, FB), lambda i: (0, i))],
        out_specs=pl.BlockSpec((FB, H // 2), lambda i: (i, 0)),
        out_shape=jax.ShapeDtypeStruct((F, H // 2), jnp.int32),
    )(W_ft)


def _mlp_body(xw_ref, xb_ref, w1_ref, b1_ref, w2_ref, b2_ref, w3_ref, b3_ref,
              wo_ref, bo_ref, o_ref):
    dn = (((1,), (1,)), ((), ()))
    xw = jnp.clip(xw_ref[...], -1.0, 1.0)
    xb = jnp.clip(xb_ref[...], -1.0, 1.0)
    h = jnp.concatenate([xw, xb], axis=1)
    h = jnp.maximum(
        lax.dot_general(h, w1_ref[...], dn, preferred_element_type=jnp.float32)
        + b1_ref[...], 0.0)
    h = jnp.maximum(
        lax.dot_general(h, w2_ref[...], dn, preferred_element_type=jnp.float32)
        + b2_ref[...], 0.0)
    h = jnp.maximum(
        lax.dot_general(h, w3_ref[...], dn, preferred_element_type=jnp.float32)
        + b3_ref[...], 0.0)
    o_ref[...] = jnp.sum(h * wo_ref[...], axis=1, keepdims=True) + bo_ref[...]


def _mlp(acc, W1, b1, W2, b2, W3, b3, W_out, b_out):
    S, H = acc.shape
    B = S // 2
    BB = 256
    NB = B // BB
    H2, H4 = W2.shape[0], W3.shape[0]
    full = lambda shape: pl.BlockSpec(shape, lambda i: (0, 0))
    return pl.pallas_call(
        _mlp_body,
        grid=(NB,),
        in_specs=[
            pl.BlockSpec((BB, H), lambda i: (i, 0)),
            pl.BlockSpec((BB, H), lambda i: (i + NB, 0)),
            full((H, 2 * H)),
            full((1, H)),
            full((H2, H)),
            full((1, H2)),
            full((H4, H2)),
            full((1, H4)),
            full((1, H4)),
            full((1, 1)),
        ],
        out_specs=pl.BlockSpec((BB, 1), lambda i: (i, 0)),
        out_shape=jax.ShapeDtypeStruct((B, 1), jnp.float32),
    )(acc, acc, W1, b1.reshape(1, -1), W2, b2.reshape(1, -1),
      W3, b3.reshape(1, -1), W_out, b_out.reshape(1, 1))


def kernel(white_indices, black_indices, W_ft, W1, b1, W2, b2, W3, b3, W_out, b_out):
    B, A = white_indices.shape
    H, F = W_ft.shape
    idx_all = jnp.concatenate([white_indices, black_indices], axis=0)
    table = _prep_table(W_ft)
    W1p = (W1.reshape(-1, 2, 2, _L, _L).swapaxes(2, 3).reshape(-1, 2 * H))
    G = 4
    seg_sum = _make_seg_sum(2 * B, A, F, H, G)
    acc = seg_sum(idx_all.reshape(2 * B // G, G * A), table)
    out = _mlp(acc, W1p, b1, W2, b2, W3, b3, W_out, b_out)
    return out[:, 0]


# speedup vs baseline: 2.1309x; 1.3750x over previous
import functools

import jax
import jax.numpy as jnp
from jax import lax
from jax.experimental import pallas as pl
from jax.experimental.pallas import tpu as pltpu
from jax.experimental.pallas import tpu_sc as plsc

_L = 16


def _make_seg_sum(S, A, F, H, G=4):
    NC, NS = 2, 16
    NW = NC * NS
    assert S % (NW * G) == 0
    SEG_W = S // NW
    GB = SEG_W // G
    GA = G * A
    KH = H // _L

    mesh = plsc.VectorSubcoreMesh(core_axis_name="c", subcore_axis_name="s")

    @functools.partial(
        pl.kernel,
        mesh=mesh,
        out_type=jax.ShapeDtypeStruct((S, H), jnp.float32),
        scratch_types=[
            pltpu.VMEM((GB, GA), jnp.int32),
            pltpu.VMEM((2, GA, H // 2), jnp.int32),
            pltpu.VMEM((SEG_W, H), jnp.float32),
            pltpu.VMEM((2 * _L * G,), jnp.float32),
            pltpu.VMEM((2 * _L * G,), jnp.int32),
            pltpu.SemaphoreType.DMA,
            pltpu.SemaphoreType.DMA,
        ],
        compiler_params=pltpu.CompilerParams(needs_layout_passes=False),
    )
    def seg_sum(idx_hbm, table_hbm, out_hbm, idx_v, rows_v, out_v, w_v, idx_s,
                sem0, sem1):
        wid = lax.axis_index("s") * NC + lax.axis_index("c")
        pltpu.sync_copy(idx_hbm.at[pl.ds(wid * GB, GB)], idx_v)

        def gather_desc(g, buf, sem):
            return pltpu.make_async_copy(
                table_hbm.at[idx_v.at[g]], rows_v.at[buf], sem)

        gather_desc(0, 0, sem0).start()
        gather_desc(1, 1, sem1).start()

        lane = lax.iota(jnp.int32, _L)
        himask = jnp.full((_L,), -65536, jnp.int32)

        def compute_group(g, buf, sem):
            for t in range(G):
                u = idx_v[g, pl.ds(t * A, _L)]
                v = idx_v[g, pl.ds(t * A + _L, _L)]
                idx_s[pl.ds(t * A, _L)] = u
                idx_s[pl.ds(t * A + _L, _L)] = v
                cnt_u = jnp.ones((_L,), jnp.int32)
                cnt_v = jnp.ones((_L,), jnp.int32)
                for s in range(1, _L):
                    rot = ((lane + s) & (_L - 1)) + t * A
                    cnt_u += (u == plsc.load_gather(idx_s, [rot])).astype(jnp.int32)
                    cnt_v += (v == plsc.load_gather(idx_s, [rot + _L])).astype(jnp.int32)
                for s in range(_L):
                    rot = ((lane + s) & (_L - 1)) + t * A
                    cnt_u += (u == plsc.load_gather(idx_s, [rot + _L])).astype(jnp.int32)
                    cnt_v += (v == plsc.load_gather(idx_s, [rot])).astype(jnp.int32)
                w_v[pl.ds(t * A, _L)] = 1.0 / cnt_u.astype(jnp.float32)
                w_v[pl.ds(t * A + _L, _L)] = 1.0 / cnt_v.astype(jnp.float32)
            gather_desc(g, buf, sem).wait()

            for t in range(G):
                def acc_body(a, acc, t=t):
                    wb = plsc.load_gather(
                        w_v, [jnp.full((_L,), t * A, jnp.int32) + a])
                    out = []
                    for k2 in range(KH // 2):
                        pair = rows_v[buf, t * A + a, pl.ds(k2 * _L, _L)]
                        lo = plsc.bitcast(pair << 16, jnp.float32)
                        hi = plsc.bitcast(pair & himask, jnp.float32)
                        out.append(acc[2 * k2] + wb * lo)
                        out.append(acc[2 * k2 + 1] + wb * hi)
                    return tuple(out)

                acc = lax.fori_loop(
                    0, A, acc_body,
                    tuple(jnp.zeros((_L,), jnp.float32) for _ in range(KH)))
                for k in range(KH):
                    out_v[g * G + t, pl.ds(k * _L, _L)] = acc[k]

            @pl.when(g + 2 < GB)
            def _():
                gather_desc(g + 2, buf, sem).start()

        def body2(i, carry):
            compute_group(2 * i, 0, sem0)
            compute_group(2 * i + 1, 1, sem1)
            return carry

        lax.fori_loop(0, GB // 2, body2, 0)
        pltpu.sync_copy(out_v, out_hbm.at[pl.ds(wid * SEG_W, SEG_W)])

    return seg_sum


def _pack_table(W_ft):
    H = W_ft.shape[0]

    def bits(z):
        return lax.bitcast_convert_type(
            z.astype(jnp.bfloat16).astype(jnp.float32), jnp.int32)

    packed = (lax.shift_right_logical(bits(W_ft[: H // 2]), 16)
              | bits(W_ft[H // 2:]))
    return packed.T


def _mlp_body(xw_ref, xb_ref, w1_ref, b1_ref, w2_ref, b2_ref, w3_ref, b3_ref,
              wo_ref, bo_ref, o_ref):
    dn = (((1,), (1,)), ((), ()))
    xw = jnp.clip(xw_ref[...], -1.0, 1.0)
    xb = jnp.clip(xb_ref[...], -1.0, 1.0)
    h = jnp.concatenate([xw, xb], axis=1)
    h = jnp.maximum(
        lax.dot_general(h, w1_ref[...], dn, preferred_element_type=jnp.float32)
        + b1_ref[...], 0.0)
    h = jnp.maximum(
        lax.dot_general(h, w2_ref[...], dn, preferred_element_type=jnp.float32)
        + b2_ref[...], 0.0)
    h = jnp.maximum(
        lax.dot_general(h, w3_ref[...], dn, preferred_element_type=jnp.float32)
        + b3_ref[...], 0.0)
    o_ref[...] = jnp.sum(h * wo_ref[...], axis=1, keepdims=True) + bo_ref[...]


def _mlp(acc, W1, b1, W2, b2, W3, b3, W_out, b_out):
    S, H = acc.shape
    B = S // 2
    BB = 256
    NB = B // BB
    H2, H4 = W2.shape[0], W3.shape[0]
    full = lambda shape: pl.BlockSpec(shape, lambda i: (0, 0))
    return pl.pallas_call(
        _mlp_body,
        grid=(NB,),
        in_specs=[
            pl.BlockSpec((BB, H), lambda i: (i, 0)),
            pl.BlockSpec((BB, H), lambda i: (i + NB, 0)),
            full((H, 2 * H)),
            full((1, H)),
            full((H2, H)),
            full((1, H2)),
            full((H4, H2)),
            full((1, H4)),
            full((1, H4)),
            full((1, 1)),
        ],
        out_specs=pl.BlockSpec((BB, 1), lambda i: (i, 0)),
        out_shape=jax.ShapeDtypeStruct((B, 1), jnp.float32),
    )(acc, acc, W1, b1.reshape(1, -1), W2, b2.reshape(1, -1),
      W3, b3.reshape(1, -1), W_out, b_out.reshape(1, 1))


def kernel(white_indices, black_indices, W_ft, W1, b1, W2, b2, W3, b3, W_out, b_out):
    B, A = white_indices.shape
    H, F = W_ft.shape
    idx_all = jnp.concatenate([white_indices, black_indices], axis=0)
    table = _pack_table(W_ft)
    W1p = (W1.reshape(-1, 2, 2, _L, _L).swapaxes(2, 3).reshape(-1, 2 * H))
    G = 4
    seg_sum = _make_seg_sum(2 * B, A, F, H, G)
    acc = seg_sum(idx_all.reshape(2 * B // G, G * A), table)
    out = _mlp(acc, W1p, b1, W2, b2, W3, b3, W_out, b_out)
    return out[:, 0]


# speedup vs baseline: 2.1376x; 1.0031x over previous
import functools

import jax
import jax.numpy as jnp
from jax import lax
from jax.experimental import pallas as pl
from jax.experimental.pallas import tpu as pltpu
from jax.experimental.pallas import tpu_sc as plsc

_L = 16


def _make_seg_sum(S, A, F, H, G=4):
    NC, NS = 2, 16
    NW = NC * NS
    assert S % (NW * G) == 0
    SEG_W = S // NW
    GB = SEG_W // G
    GA = G * A
    KH = H // _L

    mesh = plsc.VectorSubcoreMesh(core_axis_name="c", subcore_axis_name="s")

    @functools.partial(
        pl.kernel,
        mesh=mesh,
        out_type=jax.ShapeDtypeStruct((S, H), jnp.float32),
        scratch_types=[
            pltpu.VMEM((GB, GA), jnp.int32),
            pltpu.VMEM((2, GA, H // 2), jnp.int32),
            pltpu.VMEM((SEG_W, H), jnp.float32),
            pltpu.VMEM((2 * _L * G,), jnp.float32),
            pltpu.VMEM((2 * _L * G,), jnp.int32),
            pltpu.SemaphoreType.DMA,
            pltpu.SemaphoreType.DMA,
        ],
        compiler_params=pltpu.CompilerParams(needs_layout_passes=False),
    )
    def seg_sum(idx_hbm, table_hbm, out_hbm, idx_v, rows_v, out_v, w_v, idx_s,
                sem0, sem1):
        wid = lax.axis_index("s") * NC + lax.axis_index("c")
        pltpu.sync_copy(idx_hbm.at[pl.ds(wid * GB, GB)], idx_v)

        def gather_desc(g, buf, sem):
            return pltpu.make_async_copy(
                table_hbm.at[idx_v.at[g]], rows_v.at[buf], sem)

        gather_desc(0, 0, sem0).start()
        gather_desc(1, 1, sem1).start()

        lane = lax.iota(jnp.int32, _L)
        himask = jnp.full((_L,), -65536, jnp.int32)

        def compute_group(g, buf, sem):
            for t in range(G):
                u = idx_v[g, pl.ds(t * A, _L)]
                v = idx_v[g, pl.ds(t * A + _L, _L)]
                idx_s[pl.ds(t * A, _L)] = u
                idx_s[pl.ds(t * A + _L, _L)] = v
                cnt_u = jnp.ones((_L,), jnp.int32)
                cnt_v = jnp.ones((_L,), jnp.int32)
                for s in range(1, _L):
                    rot = ((lane + s) & (_L - 1)) + t * A
                    cnt_u += (u == plsc.load_gather(idx_s, [rot])).astype(jnp.int32)
                    cnt_v += (v == plsc.load_gather(idx_s, [rot + _L])).astype(jnp.int32)
                for s in range(_L):
                    rot = ((lane + s) & (_L - 1)) + t * A
                    cnt_u += (u == plsc.load_gather(idx_s, [rot + _L])).astype(jnp.int32)
                    cnt_v += (v == plsc.load_gather(idx_s, [rot])).astype(jnp.int32)
                w_v[pl.ds(t * A, _L)] = 1.0 / cnt_u.astype(jnp.float32)
                w_v[pl.ds(t * A + _L, _L)] = 1.0 / cnt_v.astype(jnp.float32)
            gather_desc(g, buf, sem).wait()

            for t in range(G):
                def acc_body(a, acc, t=t):
                    wb = plsc.load_gather(
                        w_v, [jnp.full((_L,), t * A, jnp.int32) + a])
                    out = []
                    for k2 in range(KH // 2):
                        pair = rows_v[buf, t * A + a, pl.ds(k2 * _L, _L)]
                        lo = plsc.bitcast(pair << 16, jnp.float32)
                        hi = plsc.bitcast(pair & himask, jnp.float32)
                        out.append(acc[2 * k2] + wb * lo)
                        out.append(acc[2 * k2 + 1] + wb * hi)
                    return tuple(out)

                acc = lax.fori_loop(
                    0, A, acc_body,
                    tuple(jnp.zeros((_L,), jnp.float32) for _ in range(KH)))
                for k in range(KH):
                    out_v[g * G + t, pl.ds(k * _L, _L)] = acc[k]

            @pl.when(g + 2 < GB)
            def _():
                gather_desc(g + 2, buf, sem).start()

        def body2(i, carry):
            compute_group(2 * i, 0, sem0)
            compute_group(2 * i + 1, 1, sem1)
            return carry

        lax.fori_loop(0, GB // 2, body2, 0)
        pltpu.sync_copy(out_v, out_hbm.at[pl.ds(wid * SEG_W, SEG_W)])

    return seg_sum


def _pack_table(W_ft):
    H = W_ft.shape[0]

    def bits(z):
        return lax.bitcast_convert_type(
            z.astype(jnp.bfloat16).astype(jnp.float32), jnp.int32)

    packed = (lax.shift_right_logical(bits(W_ft[: H // 2]), 16)
              | bits(W_ft[H // 2:]))
    packed = lax.optimization_barrier(packed)
    return packed.T


def _mlp_body(xw_ref, xb_ref, w1_ref, b1_ref, w2_ref, b2_ref, w3_ref, b3_ref,
              wo_ref, bo_ref, o_ref):
    dn = (((1,), (1,)), ((), ()))
    xw = jnp.clip(xw_ref[...], -1.0, 1.0)
    xb = jnp.clip(xb_ref[...], -1.0, 1.0)
    h = jnp.concatenate([xw, xb], axis=1)
    h = jnp.maximum(
        lax.dot_general(h, w1_ref[...], dn, preferred_element_type=jnp.float32)
        + b1_ref[...], 0.0)
    h = jnp.maximum(
        lax.dot_general(h, w2_ref[...], dn, preferred_element_type=jnp.float32)
        + b2_ref[...], 0.0)
    h = jnp.maximum(
        lax.dot_general(h, w3_ref[...], dn, preferred_element_type=jnp.float32)
        + b3_ref[...], 0.0)
    o_ref[...] = jnp.sum(h * wo_ref[...], axis=1, keepdims=True) + bo_ref[...]


def _mlp(acc, W1, b1, W2, b2, W3, b3, W_out, b_out):
    S, H = acc.shape
    B = S // 2
    BB = 256
    NB = B // BB
    H2, H4 = W2.shape[0], W3.shape[0]
    full = lambda shape: pl.BlockSpec(shape, lambda i: (0, 0))
    return pl.pallas_call(
        _mlp_body,
        grid=(NB,),
        in_specs=[
            pl.BlockSpec((BB, H), lambda i: (i, 0)),
            pl.BlockSpec((BB, H), lambda i: (i + NB, 0)),
            full((H, 2 * H)),
            full((1, H)),
            full((H2, H)),
            full((1, H2)),
            full((H4, H2)),
            full((1, H4)),
            full((1, H4)),
            full((1, 1)),
        ],
        out_specs=pl.BlockSpec((BB, 1), lambda i: (i, 0)),
        out_shape=jax.ShapeDtypeStruct((B, 1), jnp.float32),
    )(acc, acc, W1, b1.reshape(1, -1), W2, b2.reshape(1, -1),
      W3, b3.reshape(1, -1), W_out, b_out.reshape(1, 1))


def kernel(white_indices, black_indices, W_ft, W1, b1, W2, b2, W3, b3, W_out, b_out):
    B, A = white_indices.shape
    H, F = W_ft.shape
    idx_all = jnp.concatenate([white_indices, black_indices], axis=0)
    table = _pack_table(W_ft)
    W1p = (W1.reshape(-1, 2, 2, _L, _L).swapaxes(2, 3).reshape(-1, 2 * H))
    G = 4
    seg_sum = _make_seg_sum(2 * B, A, F, H, G)
    acc = seg_sum(idx_all.reshape(2 * B // G, G * A), table)
    out = _mlp(acc, W1p, b1, W2, b2, W3, b3, W_out, b_out)
    return out[:, 0]


# speedup vs baseline: 3.5608x; 1.6657x over previous
import functools

import jax
import jax.numpy as jnp
from jax import lax
from jax.experimental import pallas as pl
from jax.experimental.pallas import tpu as pltpu
from jax.experimental.pallas import tpu_sc as plsc

_L = 16


def _make_seg_sum(S, A, F, H, G=4):
    NC, NS = 2, 16
    NW = NC * NS
    assert S % (NW * G) == 0
    SEG_W = S // NW
    GB = SEG_W // G
    GA = G * A
    KH = H // _L

    mesh = plsc.VectorSubcoreMesh(core_axis_name="c", subcore_axis_name="s")

    @functools.partial(
        pl.kernel,
        mesh=mesh,
        out_type=jax.ShapeDtypeStruct((S, H), jnp.float32),
        scratch_types=[
            pltpu.VMEM((GB, GA), jnp.int32),
            pltpu.VMEM((2, GA, H), jnp.float32),
            pltpu.VMEM((SEG_W, H), jnp.float32),
            pltpu.VMEM((2 * _L * G,), jnp.float32),
            pltpu.VMEM((2 * _L * G,), jnp.int32),
            pltpu.SemaphoreType.DMA,
            pltpu.SemaphoreType.DMA,
        ],
        compiler_params=pltpu.CompilerParams(needs_layout_passes=False),
    )
    def seg_sum(idx_hbm, table_hbm, out_hbm, idx_v, rows_v, out_v, w_v, idx_s,
                sem0, sem1):
        wid = lax.axis_index("s") * NC + lax.axis_index("c")
        pltpu.sync_copy(idx_hbm.at[pl.ds(wid * GB, GB)], idx_v)

        def gather_desc(g, buf, sem):
            return pltpu.make_async_copy(
                table_hbm.at[idx_v.at[g]], rows_v.at[buf], sem)

        gather_desc(0, 0, sem0).start()
        gather_desc(1, 1, sem1).start()

        lane = lax.iota(jnp.int32, _L)

        def compute_group(g, buf, sem):
            for t in range(G):
                u = idx_v[g, pl.ds(t * A, _L)]
                v = idx_v[g, pl.ds(t * A + _L, _L)]
                idx_s[pl.ds(t * A, _L)] = u
                idx_s[pl.ds(t * A + _L, _L)] = v
                cnt_u = jnp.ones((_L,), jnp.int32)
                cnt_v = jnp.ones((_L,), jnp.int32)
                for s in range(1, _L):
                    rot = ((lane + s) & (_L - 1)) + t * A
                    cnt_u += (u == plsc.load_gather(idx_s, [rot])).astype(jnp.int32)
                    cnt_v += (v == plsc.load_gather(idx_s, [rot + _L])).astype(jnp.int32)
                for s in range(_L):
                    rot = ((lane + s) & (_L - 1)) + t * A
                    cnt_u += (u == plsc.load_gather(idx_s, [rot + _L])).astype(jnp.int32)
                    cnt_v += (v == plsc.load_gather(idx_s, [rot])).astype(jnp.int32)
                w_v[pl.ds(t * A, _L)] = 1.0 / cnt_u.astype(jnp.float32)
                w_v[pl.ds(t * A + _L, _L)] = 1.0 / cnt_v.astype(jnp.float32)
            gather_desc(g, buf, sem).wait()

            for t in range(G):
                def acc_body(a, acc, t=t):
                    wb = plsc.load_gather(
                        w_v, [jnp.full((_L,), t * A, jnp.int32) + a])
                    return tuple(
                        acc[k] + wb * rows_v[buf, t * A + a, pl.ds(k * _L, _L)]
                        for k in range(KH))

                acc = lax.fori_loop(
                    0, A, acc_body,
                    tuple(jnp.zeros((_L,), jnp.float32) for _ in range(KH)))
                for k in range(KH):
                    out_v[g * G + t, pl.ds(k * _L, _L)] = acc[k]

            @pl.when(g + 2 < GB)
            def _():
                gather_desc(g + 2, buf, sem).start()

        def body2(i, carry):
            compute_group(2 * i, 0, sem0)
            compute_group(2 * i + 1, 1, sem1)
            return carry

        lax.fori_loop(0, GB // 2, body2, 0)
        pltpu.sync_copy(out_v, out_hbm.at[pl.ds(wid * SEG_W, SEG_W)])

    return seg_sum


def _mlp_body(xw_ref, xb_ref, w1_ref, b1_ref, w2_ref, b2_ref, w3_ref, b3_ref,
              wo_ref, bo_ref, o_ref):
    dn = (((1,), (1,)), ((), ()))
    xw = jnp.clip(xw_ref[...], -1.0, 1.0)
    xb = jnp.clip(xb_ref[...], -1.0, 1.0)
    h = jnp.concatenate([xw, xb], axis=1)
    h = jnp.maximum(
        lax.dot_general(h, w1_ref[...], dn, preferred_element_type=jnp.float32)
        + b1_ref[...], 0.0)
    h = jnp.maximum(
        lax.dot_general(h, w2_ref[...], dn, preferred_element_type=jnp.float32)
        + b2_ref[...], 0.0)
    h = jnp.maximum(
        lax.dot_general(h, w3_ref[...], dn, preferred_element_type=jnp.float32)
        + b3_ref[...], 0.0)
    o_ref[...] = jnp.sum(h * wo_ref[...], axis=1, keepdims=True) + bo_ref[...]


def _mlp(acc, W1, b1, W2, b2, W3, b3, W_out, b_out):
    S, H = acc.shape
    B = S // 2
    BB = 256
    NB = B // BB
    H2, H4 = W2.shape[0], W3.shape[0]
    full = lambda shape: pl.BlockSpec(shape, lambda i: (0, 0))
    return pl.pallas_call(
        _mlp_body,
        grid=(NB,),
        in_specs=[
            pl.BlockSpec((BB, H), lambda i: (i, 0)),
            pl.BlockSpec((BB, H), lambda i: (i + NB, 0)),
            full((H, 2 * H)),
            full((1, H)),
            full((H2, H)),
            full((1, H2)),
            full((H4, H2)),
            full((1, H4)),
            full((1, H4)),
            full((1, 1)),
        ],
        out_specs=pl.BlockSpec((BB, 1), lambda i: (i, 0)),
        out_shape=jax.ShapeDtypeStruct((B, 1), jnp.float32),
    )(acc, acc, W1, b1.reshape(1, -1), W2, b2.reshape(1, -1),
      W3, b3.reshape(1, -1), W_out, b_out.reshape(1, 1))


def kernel(white_indices, black_indices, W_ft, W1, b1, W2, b2, W3, b3, W_out, b_out):
    B, A = white_indices.shape
    H, F = W_ft.shape
    idx_all = jnp.concatenate([white_indices, black_indices], axis=0)
    table = W_ft.T
    G = 2
    seg_sum = _make_seg_sum(2 * B, A, F, H, G)
    acc = seg_sum(idx_all.reshape(2 * B // G, G * A), table)
    out = _mlp(acc, W1, b1, W2, b2, W3, b3, W_out, b_out)
    return out[:, 0]


# speedup vs baseline: 3.6352x; 1.0209x over previous
import functools

import jax
import jax.numpy as jnp
from jax import lax
from jax.experimental import pallas as pl
from jax.experimental.pallas import tpu as pltpu
from jax.experimental.pallas import tpu_sc as plsc

_L = 16


def _make_seg_sum(S, A, F, H, G=4):
    NC, NS = 2, 16
    NW = NC * NS
    assert S % (NW * G) == 0
    SEG_W = S // NW
    GB = SEG_W // G
    GA = G * A
    KH = H // _L

    mesh = plsc.VectorSubcoreMesh(core_axis_name="c", subcore_axis_name="s")

    @functools.partial(
        pl.kernel,
        mesh=mesh,
        out_type=jax.ShapeDtypeStruct((S, H), jnp.float32),
        scratch_types=[
            pltpu.VMEM((GB, GA), jnp.int32),
            pltpu.VMEM((2, GA, H), jnp.float32),
            pltpu.VMEM((SEG_W, H), jnp.float32),
            pltpu.VMEM((2 * _L * G,), jnp.float32),
            pltpu.VMEM((2 * _L * G,), jnp.int32),
            pltpu.SemaphoreType.DMA,
            pltpu.SemaphoreType.DMA,
        ],
        compiler_params=pltpu.CompilerParams(needs_layout_passes=False),
    )
    def seg_sum(idx_hbm, table_hbm, out_hbm, idx_v, rows_v, out_v, w_v, idx_s,
                sem0, sem1):
        wid = lax.axis_index("s") * NC + lax.axis_index("c")
        pltpu.sync_copy(idx_hbm.at[pl.ds(wid * GB, GB)], idx_v)

        def gather_desc(g, buf, sem):
            return pltpu.make_async_copy(
                table_hbm.at[idx_v.at[g]], rows_v.at[buf], sem)

        gather_desc(0, 0, sem0).start()
        gather_desc(1, 1, sem1).start()

        lane = lax.iota(jnp.int32, _L)

        def compute_group(g, buf, sem):
            for t in range(G):
                u = idx_v[g, pl.ds(t * A, _L)]
                v = idx_v[g, pl.ds(t * A + _L, _L)]
                idx_s[pl.ds(t * A, _L)] = u
                idx_s[pl.ds(t * A + _L, _L)] = v
                cnt_u = jnp.ones((_L,), jnp.int32)
                cnt_v = jnp.ones((_L,), jnp.int32)
                for s in range(1, _L):
                    rot = ((lane + s) & (_L - 1)) + t * A
                    cnt_u += (u == plsc.load_gather(idx_s, [rot])).astype(jnp.int32)
                    cnt_v += (v == plsc.load_gather(idx_s, [rot + _L])).astype(jnp.int32)
                for s in range(_L):
                    rot = ((lane + s) & (_L - 1)) + t * A
                    cnt_u += (u == plsc.load_gather(idx_s, [rot + _L])).astype(jnp.int32)
                    cnt_v += (v == plsc.load_gather(idx_s, [rot])).astype(jnp.int32)
                w_v[pl.ds(t * A, _L)] = 1.0 / cnt_u.astype(jnp.float32)
                w_v[pl.ds(t * A + _L, _L)] = 1.0 / cnt_v.astype(jnp.float32)
            gather_desc(g, buf, sem).wait()

            for t in range(G):
                def acc_body(a, acc, t=t):
                    wb = plsc.load_gather(
                        w_v, [jnp.full((_L,), t * A, jnp.int32) + a])
                    return tuple(
                        acc[k] + wb * rows_v[buf, t * A + a, pl.ds(k * _L, _L)]
                        for k in range(KH))

                acc = lax.fori_loop(
                    0, A, acc_body,
                    tuple(jnp.zeros((_L,), jnp.float32) for _ in range(KH)))
                for k in range(KH):
                    out_v[g * G + t, pl.ds(k * _L, _L)] = acc[k]

            @pl.when(g + 2 < GB)
            def _():
                gather_desc(g + 2, buf, sem).start()

        def body2(i, carry):
            compute_group(2 * i, 0, sem0)
            compute_group(2 * i + 1, 1, sem1)
            return carry

        lax.fori_loop(0, GB // 2, body2, 0)
        pltpu.sync_copy(out_v, out_hbm.at[pl.ds(wid * SEG_W, SEG_W)])

    return seg_sum


def _mlp_body(xw_ref, xb_ref, w1_ref, b1_ref, w2_ref, b2_ref, w3_ref, b3_ref,
              wo_ref, bo_ref, o_ref):
    dn = (((1,), (1,)), ((), ()))
    xw = jnp.clip(xw_ref[...], -1.0, 1.0)
    xb = jnp.clip(xb_ref[...], -1.0, 1.0)
    h = jnp.concatenate([xw, xb], axis=1)
    h = jnp.maximum(
        lax.dot_general(h, w1_ref[...], dn, preferred_element_type=jnp.float32)
        + b1_ref[...], 0.0)
    h = jnp.maximum(
        lax.dot_general(h, w2_ref[...], dn, preferred_element_type=jnp.float32)
        + b2_ref[...], 0.0)
    h = jnp.maximum(
        lax.dot_general(h, w3_ref[...], dn, preferred_element_type=jnp.float32)
        + b3_ref[...], 0.0)
    o_ref[...] = jnp.sum(h * wo_ref[...], axis=1, keepdims=True) + bo_ref[...]


def _mlp(acc, W1, b1, W2, b2, W3, b3, W_out, b_out):
    S, H = acc.shape
    B = S // 2
    BB = 1024
    NB = B // BB
    H2, H4 = W2.shape[0], W3.shape[0]
    full = lambda shape: pl.BlockSpec(shape, lambda i: (0, 0))
    return pl.pallas_call(
        _mlp_body,
        grid=(NB,),
        in_specs=[
            pl.BlockSpec((BB, H), lambda i: (i, 0)),
            pl.BlockSpec((BB, H), lambda i: (i + NB, 0)),
            full((H, 2 * H)),
            full((1, H)),
            full((H2, H)),
            full((1, H2)),
            full((H4, H2)),
            full((1, H4)),
            full((1, H4)),
            full((1, 1)),
        ],
        out_specs=pl.BlockSpec((BB, 1), lambda i: (i, 0)),
        out_shape=jax.ShapeDtypeStruct((B, 1), jnp.float32),
    )(acc, acc, W1, b1.reshape(1, -1), W2, b2.reshape(1, -1),
      W3, b3.reshape(1, -1), W_out, b_out.reshape(1, 1))


def kernel(white_indices, black_indices, W_ft, W1, b1, W2, b2, W3, b3, W_out, b_out):
    B, A = white_indices.shape
    H, F = W_ft.shape
    idx_all = jnp.concatenate([white_indices, black_indices], axis=0)
    table = W_ft.T
    G = 2
    seg_sum = _make_seg_sum(2 * B, A, F, H, G)
    acc = seg_sum(idx_all.reshape(2 * B // G, G * A), table)
    out = _mlp(acc, W1, b1, W2, b2, W3, b3, W_out, b_out)
    return out[:, 0]


# speedup vs baseline: 3.6431x; 1.0022x over previous
import functools

import jax
import jax.numpy as jnp
from jax import lax
from jax.experimental import pallas as pl
from jax.experimental.pallas import tpu as pltpu
from jax.experimental.pallas import tpu_sc as plsc

_L = 16


def _make_seg_sum(S, A, F, H, G=4):
    NC, NS = 2, 16
    NW = NC * NS
    assert S % (NW * G) == 0
    SEG_W = S // NW
    GB = SEG_W // G
    GA = G * A
    KH = H // _L

    mesh = plsc.VectorSubcoreMesh(core_axis_name="c", subcore_axis_name="s")

    @functools.partial(
        pl.kernel,
        mesh=mesh,
        out_type=jax.ShapeDtypeStruct((S, H), jnp.float32),
        scratch_types=[
            pltpu.VMEM((GB, GA), jnp.int32),
            pltpu.VMEM((2, GA, H), jnp.float32),
            pltpu.VMEM((SEG_W, H), jnp.float32),
            pltpu.SemaphoreType.DMA,
            pltpu.SemaphoreType.DMA,
        ],
        compiler_params=pltpu.CompilerParams(needs_layout_passes=False),
    )
    def seg_sum(idx_hbm, table_hbm, out_hbm, idx_v, rows_v, out_v, sem0, sem1):
        wid = lax.axis_index("s") * NC + lax.axis_index("c")
        pltpu.sync_copy(idx_hbm.at[pl.ds(wid * GB, GB)], idx_v)

        def gather_desc(g, buf, sem):
            return pltpu.make_async_copy(
                table_hbm.at[idx_v.at[g]], rows_v.at[buf], sem)

        gather_desc(0, 0, sem0).start()
        gather_desc(1, 1, sem1).start()

        lane = lax.iota(jnp.int32, _L)

        gdn = lax.GatherDimensionNumbers(
            offset_dims=(), collapsed_slice_dims=(0,), start_index_map=(0,))

        def take(x, i):
            return lax.gather(x, i[:, None], gdn, slice_sizes=(1,),
                              mode=lax.GatherScatterMode.PROMISE_IN_BOUNDS)

        def compute_group(g, buf, sem):
            wlist = []
            for t in range(G):
                u = idx_v[g, pl.ds(t * A, _L)]
                v = idx_v[g, pl.ds(t * A + _L, _L)]
                cnt_u = jnp.ones((_L,), jnp.int32)
                cnt_v = jnp.ones((_L,), jnp.int32)
                for s in range(1, _L):
                    rot = (lane + s) & (_L - 1)
                    cnt_u += (u == take(u, rot)).astype(jnp.int32)
                    cnt_v += (v == take(v, rot)).astype(jnp.int32)
                for s in range(_L):
                    rot = (lane + s) & (_L - 1)
                    cnt_u += (u == take(v, rot)).astype(jnp.int32)
                    cnt_v += (v == take(u, rot)).astype(jnp.int32)
                wlist.append((1.0 / cnt_u.astype(jnp.float32),
                              1.0 / cnt_v.astype(jnp.float32)))
            gather_desc(g, buf, sem).wait()

            for t in range(G):
                wu, wv = wlist[t]

                def acc_body(a, acc, t=t, wu=wu, wv=wv):
                    fa = jnp.full((_L,), 0, jnp.int32) + (a & (_L - 1))
                    wb = jnp.where(a < _L, take(wu, fa), take(wv, fa))
                    return tuple(
                        acc[k] + wb * rows_v[buf, t * A + a, pl.ds(k * _L, _L)]
                        for k in range(KH))

                acc = lax.fori_loop(
                    0, A, acc_body,
                    tuple(jnp.zeros((_L,), jnp.float32) for _ in range(KH)))
                for k in range(KH):
                    out_v[g * G + t, pl.ds(k * _L, _L)] = acc[k]

            @pl.when(g + 2 < GB)
            def _():
                gather_desc(g + 2, buf, sem).start()

        def body2(i, carry):
            compute_group(2 * i, 0, sem0)
            compute_group(2 * i + 1, 1, sem1)
            return carry

        lax.fori_loop(0, GB // 2, body2, 0)
        pltpu.sync_copy(out_v, out_hbm.at[pl.ds(wid * SEG_W, SEG_W)])

    return seg_sum


def _mlp_body(xw_ref, xb_ref, w1_ref, b1_ref, w2_ref, b2_ref, w3_ref, b3_ref,
              wo_ref, bo_ref, o_ref):
    dn = (((1,), (1,)), ((), ()))
    dot = functools.partial(
        lax.dot_general, dimension_numbers=dn,
        preferred_element_type=jnp.float32)
    xw = jnp.clip(xw_ref[...], -1.0, 1.0)
    xb = jnp.clip(xb_ref[...], -1.0, 1.0)
    h = jnp.concatenate([xw, xb], axis=1)
    h = jnp.maximum(dot(h, w1_ref[...]) + b1_ref[...], 0.0)
    h = jnp.maximum(dot(h, w2_ref[...]) + b2_ref[...], 0.0)
    h = jnp.maximum(dot(h, w3_ref[...]) + b3_ref[...], 0.0)
    o_ref[...] = jnp.sum(h * wo_ref[...], axis=1, keepdims=True) + bo_ref[...]


def _mlp(acc, W1, b1, W2, b2, W3, b3, W_out, b_out):
    S, H = acc.shape
    B = S // 2
    BB = 1024
    NB = B // BB
    H2, H4 = W2.shape[0], W3.shape[0]
    full = lambda shape: pl.BlockSpec(shape, lambda i: (0, 0))
    return pl.pallas_call(
        _mlp_body,
        grid=(NB,),
        in_specs=[
            pl.BlockSpec((BB, H), lambda i: (i, 0)),
            pl.BlockSpec((BB, H), lambda i: (i + NB, 0)),
            full((H, 2 * H)),
            full((1, H)),
            full((H2, H)),
            full((1, H2)),
            full((H4, H2)),
            full((1, H4)),
            full((1, H4)),
            full((1, 1)),
        ],
        out_specs=pl.BlockSpec((BB, 1), lambda i: (i, 0)),
        out_shape=jax.ShapeDtypeStruct((B, 1), jnp.float32),
    )(acc, acc, W1, b1.reshape(1, -1), W2, b2.reshape(1, -1),
      W3, b3.reshape(1, -1), W_out, b_out.reshape(1, 1))


def kernel(white_indices, black_indices, W_ft, W1, b1, W2, b2, W3, b3, W_out, b_out):
    B, A = white_indices.shape
    H, F = W_ft.shape
    idx_all = jnp.concatenate([white_indices, black_indices], axis=0)
    table = W_ft.T
    G = 2
    seg_sum = _make_seg_sum(2 * B, A, F, H, G)
    acc = seg_sum(idx_all.reshape(2 * B // G, G * A), table)
    out = _mlp(acc, W1, b1, W2, b2, W3, b3, W_out, b_out)
    return out[:, 0]
